# baseline (speedup 1.0000x reference)
import jax
import jax.numpy as jnp
from jax.experimental import pallas as pl
from jax.experimental.pallas import tpu as pltpu

_K = 64
_BISECT_ITERS = 31


def _proj_body(x_ref, w_ref, b_ref, o_ref):
    o_ref[...] = (
        jnp.dot(x_ref[...], w_ref[...], preferred_element_type=jnp.float32)
        + b_ref[...]
    )


def _grav_body(xs_ref, spT_ref, pp_ref, ppT_ref, xb_ref,
               wo1_ref, wo2_ref, wo3_ref, bo_ref, out_ref, fmax_s):
    pts = xs_ref[0]
    spT = spT_ref[0]
    r = pts.shape[0]

    sq_i = jnp.sum(pts * pts, axis=1, keepdims=True)
    sq_j = jnp.sum(spT * spT, axis=0, keepdims=True)
    d2 = sq_i + sq_j - 2.0 * jnp.dot(pts, spT,
                                     preferred_element_type=jnp.float32)
    d2c = jnp.maximum(d2, 0.0)

    d2i = d2c.view(jnp.int32)
    lo0 = jnp.full((r, 1), -1, dtype=jnp.int32)
    hi0 = jnp.max(d2i, axis=1, keepdims=True) + 1

    def body(_, carry):
        lo, hi = carry
        mid = lo + jax.lax.div(hi - lo, 2)
        cnt = jnp.sum((d2i <= mid).astype(jnp.float32), axis=1, keepdims=True)
        pred = cnt >= float(_K)
        return jnp.where(pred, lo, mid), jnp.where(pred, mid, hi)

    _, thr = jax.lax.fori_loop(0, _BISECT_ITERS, body, (lo0, hi0))

    mask = d2i <= thr
    w = jnp.where(mask, jnp.exp(-10.0 * d2c), 0.0)

    fmean = jnp.dot(w, pp_ref[0], preferred_element_type=jnp.float32) * (1.0 / _K)

    neg = jnp.where(mask, 0.0, -1e30)
    ppT = ppT_ref[0]
    pd = ppT.shape[0]
    for dd in range(pd):
        cand = jnp.max(w * ppT[dd:dd + 1, :] + neg, axis=1)
        fmax_s[:, dd:dd + 1] = cand[:, None]
    fmax = fmax_s[...]

    h = (
        jnp.dot(xb_ref[0], wo1_ref[...], preferred_element_type=jnp.float32)
        + jnp.dot(fmean, wo2_ref[...], preferred_element_type=jnp.float32)
        + jnp.dot(fmax, wo3_ref[...], preferred_element_type=jnp.float32)
        + bo_ref[...]
    )
    out_ref[0] = jnp.maximum(h, 0.0)


def kernel(x, row_splits, Ws, bs, Wp, bp, Wo, bo):
    n, din = x.shape
    nseg = row_splits.shape[0] - 1
    seg = n // nseg
    sd = Ws.shape[1]
    pd = Wp.shape[1]
    dout = Wo.shape[1]

    r = 500 if seg % 500 == 0 else seg
    bps = seg // r
    nb = n // r

    wsp = jnp.concatenate([Ws, Wp], axis=1)
    bsp = jnp.concatenate([bs, bp], axis=0).reshape(1, sd + pd)
    pr = 2000 if n % 2000 == 0 else n
    sp = pl.pallas_call(
        _proj_body,
        grid=(n // pr,),
        in_specs=[
            pl.BlockSpec((pr, din), lambda i: (i, 0)),
            pl.BlockSpec((din, sd + pd), lambda i: (0, 0)),
            pl.BlockSpec((1, sd + pd), lambda i: (0, 0)),
        ],
        out_specs=pl.BlockSpec((pr, sd + pd), lambda i: (i, 0)),
        out_shape=jax.ShapeDtypeStruct((n, sd + pd), jnp.float32),
        compiler_params=pltpu.CompilerParams(
            dimension_semantics=("parallel",)),
    )(x, wsp, bsp)

    space = sp[:, :sd]
    prop = sp[:, sd:sd + pd]

    xs = space.reshape(nb, r, sd)
    spT = space.reshape(nseg, seg, sd).transpose(0, 2, 1)
    pp = prop.reshape(nseg, seg, pd)
    ppT = pp.transpose(0, 2, 1)
    xb = x.reshape(nb, r, din)
    wo1 = Wo[:din]
    wo2 = Wo[din:din + pd]
    wo3 = Wo[din + pd:]
    bo2 = bo.reshape(1, dout)

    out = pl.pallas_call(
        _grav_body,
        grid=(nb,),
        in_specs=[
            pl.BlockSpec((1, r, sd), lambda b: (b, 0, 0)),
            pl.BlockSpec((1, sd, seg), lambda b: (b // bps, 0, 0)),
            pl.BlockSpec((1, seg, pd), lambda b: (b // bps, 0, 0)),
            pl.BlockSpec((1, pd, seg), lambda b: (b // bps, 0, 0)),
            pl.BlockSpec((1, r, din), lambda b: (b, 0, 0)),
            pl.BlockSpec((din, dout), lambda b: (0, 0)),
            pl.BlockSpec((pd, dout), lambda b: (0, 0)),
            pl.BlockSpec((pd, dout), lambda b: (0, 0)),
            pl.BlockSpec((1, dout), lambda b: (0, 0)),
        ],
        out_specs=pl.BlockSpec((1, r, dout), lambda b: (b, 0, 0)),
        out_shape=jax.ShapeDtypeStruct((nb, r, dout), jnp.float32),
        scratch_shapes=[pltpu.VMEM((r, pd), jnp.float32)],
        compiler_params=pltpu.CompilerParams(
            dimension_semantics=("parallel",)),
    )(xs, spT, pp, ppT, xb, wo1, wo2, wo3, bo2)

    return out.reshape(n, dout)


# speedup vs baseline: 1.1576x; 1.1576x over previous
import jax
import jax.numpy as jnp
from jax.experimental import pallas as pl
from jax.experimental.pallas import tpu as pltpu

_K = 64
_BISECT_ITERS = 31


def _proj_body(x_ref, w_ref, b_ref, o_ref):
    o_ref[...] = (
        jnp.dot(x_ref[...], w_ref[...], preferred_element_type=jnp.float32)
        + b_ref[...]
    )


def _grav_body(xs_ref, spT_ref, pp_ref, ppT_ref, xb_ref,
               wo1_ref, wo2_ref, wo3_ref, bo_ref, out_ref, fmax_s):
    pts = xs_ref[0]
    spT = spT_ref[0]
    r = pts.shape[0]

    sq_i = jnp.sum(pts * pts, axis=1, keepdims=True)
    sq_j = jnp.sum(spT * spT, axis=0, keepdims=True)
    d2 = sq_i + sq_j - 2.0 * jnp.dot(pts, spT,
                                     preferred_element_type=jnp.float32)
    d2c = jnp.maximum(d2, 0.0)

    d2i = d2c.view(jnp.int32)
    lo0 = jnp.full((r, 1), -1, dtype=jnp.int32)
    hi0 = jnp.max(d2i, axis=1, keepdims=True) + 1

    def body(_, carry):
        lo, hi = carry
        mid = lo + jax.lax.div(hi - lo, 2)
        cnt = jnp.sum((d2i <= mid).astype(jnp.float32), axis=1, keepdims=True)
        pred = cnt >= float(_K)
        return jnp.where(pred, lo, mid), jnp.where(pred, mid, hi)

    _, thr = jax.lax.fori_loop(0, _BISECT_ITERS, body, (lo0, hi0))

    mask = d2i <= thr
    w = jnp.where(mask, jnp.exp(-10.0 * d2c), 0.0)

    fmean = jnp.dot(w, pp_ref[0], preferred_element_type=jnp.float32) * (1.0 / _K)

    neg = jnp.where(mask, 0.0, -1e30)
    ppT = ppT_ref[0]
    pd = ppT.shape[0]
    for dd in range(pd):
        cand = jnp.max(w * ppT[dd:dd + 1, :] + neg, axis=1)
        fmax_s[:, dd:dd + 1] = cand[:, None]
    fmax = fmax_s[...]

    h = (
        jnp.dot(xb_ref[0], wo1_ref[...], preferred_element_type=jnp.float32)
        + jnp.dot(fmean, wo2_ref[...], preferred_element_type=jnp.float32)
        + jnp.dot(fmax, wo3_ref[...], preferred_element_type=jnp.float32)
        + bo_ref[...]
    )
    out_ref[0] = jnp.maximum(h, 0.0)


def kernel(x, row_splits, Ws, bs, Wp, bp, Wo, bo):
    n, din = x.shape
    nseg = row_splits.shape[0] - 1
    seg = n // nseg
    sd = Ws.shape[1]
    pd = Wp.shape[1]
    dout = Wo.shape[1]

    r = 250 if seg % 250 == 0 else seg
    bps = seg // r
    nb = n // r

    wsp = jnp.concatenate([Ws, Wp], axis=1)
    bsp = jnp.concatenate([bs, bp], axis=0).reshape(1, sd + pd)
    pr = 2000 if n % 2000 == 0 else n
    sp = pl.pallas_call(
        _proj_body,
        grid=(n // pr,),
        in_specs=[
            pl.BlockSpec((pr, din), lambda i: (i, 0)),
            pl.BlockSpec((din, sd + pd), lambda i: (0, 0)),
            pl.BlockSpec((1, sd + pd), lambda i: (0, 0)),
        ],
        out_specs=pl.BlockSpec((pr, sd + pd), lambda i: (i, 0)),
        out_shape=jax.ShapeDtypeStruct((n, sd + pd), jnp.float32),
        compiler_params=pltpu.CompilerParams(
            dimension_semantics=("parallel",)),
    )(x, wsp, bsp)

    space = sp[:, :sd]
    prop = sp[:, sd:sd + pd]

    xs = space.reshape(nb, r, sd)
    spT = space.reshape(nseg, seg, sd).transpose(0, 2, 1)
    pp = prop.reshape(nseg, seg, pd)
    ppT = pp.transpose(0, 2, 1)
    xb = x.reshape(nb, r, din)
    wo1 = Wo[:din]
    wo2 = Wo[din:din + pd]
    wo3 = Wo[din + pd:]
    bo2 = bo.reshape(1, dout)

    out = pl.pallas_call(
        _grav_body,
        grid=(nb,),
        in_specs=[
            pl.BlockSpec((1, r, sd), lambda b: (b, 0, 0)),
            pl.BlockSpec((1, sd, seg), lambda b: (b // bps, 0, 0)),
            pl.BlockSpec((1, seg, pd), lambda b: (b // bps, 0, 0)),
            pl.BlockSpec((1, pd, seg), lambda b: (b // bps, 0, 0)),
            pl.BlockSpec((1, r, din), lambda b: (b, 0, 0)),
            pl.BlockSpec((din, dout), lambda b: (0, 0)),
            pl.BlockSpec((pd, dout), lambda b: (0, 0)),
            pl.BlockSpec((pd, dout), lambda b: (0, 0)),
            pl.BlockSpec((1, dout), lambda b: (0, 0)),
        ],
        out_specs=pl.BlockSpec((1, r, dout), lambda b: (b, 0, 0)),
        out_shape=jax.ShapeDtypeStruct((nb, r, dout), jnp.float32),
        scratch_shapes=[pltpu.VMEM((r, pd), jnp.float32)],
        compiler_params=pltpu.CompilerParams(
            dimension_semantics=("parallel",)),
    )(xs, spT, pp, ppT, xb, wo1, wo2, wo3, bo2)

    return out.reshape(n, dout)
